# Initial kernel scaffold; baseline (speedup 1.0000x reference)
import jax
import jax.numpy as jnp
from jax.experimental import pallas as pl


def kernel(x_card, x_merchant, x_device, edge_pays, edge_uses, edge_seen_at, W_card, b_card, W_merchant, b_merchant, W_device, b_device, W_mlp0, b_mlp0, ln0_g, ln0_b, W_mlp1, b_mlp1, ln1_g, ln1_b):
    raise NotImplementedError("write your pallas kernel here")


# speedup vs baseline: 2.2282x; 2.2282x over previous
import functools
import math

import jax
import jax.numpy as jnp
from jax import lax
from jax.experimental import pallas as pl
from jax.experimental.pallas import tpu as pltpu, tpu_sc as plsc

N_CARD = 50000
N_MERCH = 30000
N_DEV = 20000
N = N_CARD + N_MERCH + N_DEV
HIDDEN = 128
OUT = 64
LN_EPS = 1e-5

E = 2 * (150000 + 100000 + 50000)
EB = 128
NSC = 2
NTILE = 16
RPT = 304
NSEG = 19
RSEG = RPT // NSEG
NB = RPT * NTILE
EP = NB * EB
NDUMP = 128
NROWS = 100224
ZROWS = NROWS // NTILE
ZBUF = 696

_SQRT_HALF = 1.0 / math.sqrt(2.0)


def _enc_body(x_ref, w_ref, b_ref, wb_ref, x_out, y_out):
    xb = jnp.maximum(
        jnp.dot(x_ref[...], w_ref[...], preferred_element_type=jnp.float32)
        + b_ref[...], 0.0)
    x_out[...] = xb
    y_out[...] = jnp.dot(xb, wb_ref[...], preferred_element_type=jnp.float32)


def _encode(x, w, b, w_bot, rows, blk):
    grid = rows // blk
    fin = x.shape[1]
    fout = w_bot.shape[1]
    return pl.pallas_call(
        _enc_body,
        grid=(grid,),
        in_specs=[
            pl.BlockSpec((blk, fin), lambda i: (i, 0)),
            pl.BlockSpec((fin, HIDDEN), lambda i: (0, 0)),
            pl.BlockSpec((1, HIDDEN), lambda i: (0, 0)),
            pl.BlockSpec((HIDDEN, fout), lambda i: (0, 0)),
        ],
        out_specs=[
            pl.BlockSpec((blk, HIDDEN), lambda i: (i, 0)),
            pl.BlockSpec((blk, fout), lambda i: (i, 0)),
        ],
        out_shape=[
            jax.ShapeDtypeStruct((rows, HIDDEN), jnp.float32),
            jax.ShapeDtypeStruct((rows, fout), jnp.float32),
        ],
    )(x, w, b.reshape(1, HIDDEN), w_bot)


def _gelu_ln(h, g_ref, be_ref):
    h = 0.5 * h * (1.0 + lax.erf(h * _SQRT_HALF))
    mu = jnp.mean(h, axis=-1, keepdims=True)
    var = jnp.mean((h - mu) ** 2, axis=-1, keepdims=True)
    return (h - mu) * lax.rsqrt(var + LN_EPS) * g_ref[...] + be_ref[...]


def _layer_mid_body(x_ref, agg_ref, deg_ref, wt_ref, b_ref, g_ref, be_ref,
                    wb_ref, x1_out, y1_out):
    deg = deg_ref[...]
    degs = deg[:, 0:1] + deg[:, 16:17]
    inv = 1.0 / jnp.maximum(degs, 1.0)
    h = (jnp.dot(x_ref[...], wt_ref[...], preferred_element_type=jnp.float32)
         + agg_ref[...] * inv + b_ref[...])
    x1 = _gelu_ln(h, g_ref, be_ref)
    x1_out[...] = x1
    y1_out[...] = jnp.dot(x1, wb_ref[...], preferred_element_type=jnp.float32)


def _layer_mid(x, agg, deg, w_top, b, g, be, w_bot, blk):
    grid = N // blk
    return pl.pallas_call(
        _layer_mid_body,
        grid=(grid,),
        in_specs=[
            pl.BlockSpec((blk, HIDDEN), lambda i: (i, 0)),
            pl.BlockSpec((blk, HIDDEN), lambda i: (i, 0)),
            pl.BlockSpec((blk, 32), lambda i: (i, 0)),
            pl.BlockSpec((HIDDEN, HIDDEN), lambda i: (0, 0)),
            pl.BlockSpec((1, HIDDEN), lambda i: (0, 0)),
            pl.BlockSpec((1, HIDDEN), lambda i: (0, 0)),
            pl.BlockSpec((1, HIDDEN), lambda i: (0, 0)),
            pl.BlockSpec((HIDDEN, OUT), lambda i: (0, 0)),
        ],
        out_specs=[
            pl.BlockSpec((blk, HIDDEN), lambda i: (i, 0)),
            pl.BlockSpec((blk, OUT), lambda i: (i, 0)),
        ],
        out_shape=[
            jax.ShapeDtypeStruct((N, HIDDEN), jnp.float32),
            jax.ShapeDtypeStruct((N, OUT), jnp.float32),
        ],
    )(x, agg, deg, w_top, b.reshape(1, -1), g.reshape(1, -1),
      be.reshape(1, -1), w_bot)


def _layer_out_body(x_ref, agg_ref, deg_ref, wt_ref, b_ref, g_ref, be_ref,
                    out_ref):
    deg = deg_ref[...]
    degs = deg[:, 0:1] + deg[:, 16:17]
    inv = 1.0 / jnp.maximum(degs, 1.0)
    h = (jnp.dot(x_ref[...], wt_ref[...], preferred_element_type=jnp.float32)
         + agg_ref[...] * inv + b_ref[...])
    out_ref[...] = _gelu_ln(h, g_ref, be_ref)


def _layer_out(x, agg, deg, w_top, b, g, be, blk):
    grid = N // blk
    return pl.pallas_call(
        _layer_out_body,
        grid=(grid,),
        in_specs=[
            pl.BlockSpec((blk, HIDDEN), lambda i: (i, 0)),
            pl.BlockSpec((blk, OUT), lambda i: (i, 0)),
            pl.BlockSpec((blk, 32), lambda i: (i, 0)),
            pl.BlockSpec((HIDDEN, OUT), lambda i: (0, 0)),
            pl.BlockSpec((1, OUT), lambda i: (0, 0)),
            pl.BlockSpec((1, OUT), lambda i: (0, 0)),
            pl.BlockSpec((1, OUT), lambda i: (0, 0)),
        ],
        out_specs=pl.BlockSpec((blk, OUT), lambda i: (i, 0)),
        out_shape=jax.ShapeDtypeStruct((N, OUT), jnp.float32),
    )(x, agg, deg, w_top, b.reshape(1, -1), g.reshape(1, -1),
      be.reshape(1, -1))


def _zero_slice(acc, zbuf, sid):
    base = sid * ZROWS
    for t in range(ZROWS // ZBUF):
        pltpu.sync_copy(zbuf, acc.at[pl.ds(base + t * ZBUF, ZBUF)])
    rem = ZROWS % ZBUF
    if rem:
        pltpu.sync_copy(zbuf.at[pl.ds(0, rem)],
                        acc.at[pl.ds(base + (ZROWS // ZBUF) * ZBUF, rem)])


def _make_sc_agg(nchunk, with_deg):
    kpc = nchunk // NSC
    out_type = [jax.ShapeDtypeStruct((NROWS, nchunk, 16), jnp.float32)]
    if with_deg:
        out_type.append(jax.ShapeDtypeStruct((NROWS, 2, 16), jnp.float32))

    mesh = plsc.VectorSubcoreMesh(core_axis_name="c", subcore_axis_name="s")

    @functools.partial(
        pl.kernel,
        out_type=out_type,
        mesh=mesh,
        compiler_params=pltpu.CompilerParams(use_tc_tiling_on_sc=False),
        scratch_types=[
            pltpu.VMEM_SHARED((NROWS, 16), jnp.float32),
            pltpu.VMEM((RSEG, EB), jnp.int32),
            pltpu.VMEM((RSEG, EB), jnp.int32),
            pltpu.VMEM((EB, 16), jnp.float32),
            pltpu.VMEM((ZBUF, 16), jnp.float32),
        ],
    )
    def sc_agg(table, idx, dst, *rest):
        if with_deg:
            agg_out, deg_out, acc, idxv, dstv, rows, zbuf = rest
        else:
            agg_out, acc, idxv, dstv, rows, zbuf = rest
        cid = lax.axis_index("c")
        sid = lax.axis_index("s")

        def _zb(i, _):
            zbuf[i, :] = jnp.zeros((16,), jnp.float32)
            return 0
        lax.fori_loop(0, ZBUF, _zb, 0)

        def _chunk(k):
            _zero_slice(acc, zbuf, sid)
            plsc.subcore_barrier()
            for s in range(NSEG):
                base = s * (NB // NSEG) + sid * RSEG
                pltpu.sync_copy(idx.at[pl.ds(k * NB + base, RSEG)], idxv)
                pltpu.sync_copy(dst.at[pl.ds(base, RSEG)], dstv)

                def _edge(j, _):
                    pltpu.sync_copy(table.at[idxv.at[j]], rows)
                    pltpu.sync_copy(rows, acc.at[dstv.at[j]], add=True)
                    return 0
                lax.fori_loop(0, RSEG, _edge, 0)
            plsc.subcore_barrier()
            pltpu.sync_copy(acc.at[pl.ds(sid * ZROWS, ZROWS)],
                            agg_out.at[pl.ds(sid * ZROWS, ZROWS), k])
            plsc.subcore_barrier()

        for cc in range(NSC):
            @pl.when(cid == cc)
            def _core_chunks(cc=cc):
                for kk in range(kpc):
                    _chunk(cc * kpc + kk)

        if with_deg:
            def _ob(i, _):
                rows[i, :] = jnp.ones((16,), jnp.float32)
                return 0
            lax.fori_loop(0, EB, _ob, 0)
            _zero_slice(acc, zbuf, sid)
            plsc.subcore_barrier()
            half = RSEG // NSC

            def _deg_segs(cc):
                for s in range(NSEG):
                    base = s * (NB // NSEG) + sid * RSEG
                    pltpu.sync_copy(dst.at[pl.ds(base, RSEG)], dstv)

                    def _dedge(j, _):
                        pltpu.sync_copy(rows, acc.at[dstv.at[j]], add=True)
                        return 0
                    lax.fori_loop(cc * half, (cc + 1) * half, _dedge, 0)
                plsc.subcore_barrier()
                pltpu.sync_copy(acc.at[pl.ds(sid * ZROWS, ZROWS)],
                                deg_out.at[pl.ds(sid * ZROWS, ZROWS), cc])

            for cc in range(NSC):
                @pl.when(cid == cc)
                def _core_deg(cc=cc):
                    _deg_segs(cc)

    return sc_agg


_make_sc_agg = functools.lru_cache(None)(_make_sc_agg)


def _sc_agg8(*args):
    return _make_sc_agg(8, True)(*args)


def _sc_agg4(*args):
    return _make_sc_agg(4, False)(*args)


def kernel(x_card, x_merchant, x_device, edge_pays, edge_uses, edge_seen_at,
           W_card, b_card, W_merchant, b_merchant, W_device, b_device,
           W_mlp0, b_mlp0, ln0_g, ln0_b, W_mlp1, b_mlp1, ln1_g, ln1_b):
    mo = N_CARD
    do = N_CARD + N_MERCH
    src = jnp.concatenate([
        edge_pays[0], edge_pays[1] + mo, edge_uses[0], edge_uses[1] + do,
        edge_seen_at[0] + do, edge_seen_at[1] + mo])
    dst = jnp.concatenate([
        edge_pays[1] + mo, edge_pays[0], edge_uses[1] + do, edge_uses[0],
        edge_seen_at[1] + mo, edge_seen_at[0] + do])
    npad = EP - E
    pad_lanes = jnp.arange(npad, dtype=src.dtype)
    src_p = jnp.concatenate([src, pad_lanes % NDUMP])
    dst_p = jnp.concatenate([dst, N + (pad_lanes % NDUMP)])
    dst2d = dst_p.reshape(NB, EB)
    ks8 = jnp.arange(8, dtype=src.dtype)[:, None]
    idx8 = (src_p[None, :] * 8 + ks8).reshape(8 * NB, EB)
    ks4 = jnp.arange(4, dtype=src.dtype)[:, None]
    idx4 = (src_p[None, :] * 4 + ks4).reshape(4 * NB, EB)

    w0_top, w0_bot = W_mlp0[:HIDDEN], W_mlp0[HIDDEN:]
    w1_top, w1_bot = W_mlp1[:HIDDEN], W_mlp1[HIDDEN:]

    xc, yc = _encode(x_card, W_card, b_card, w0_bot, N_CARD, 1000)
    xm, ym = _encode(x_merchant, W_merchant, b_merchant, w0_bot, N_MERCH, 1000)
    xd, yd = _encode(x_device, W_device, b_device, w0_bot, N_DEV, 1000)
    x = jnp.concatenate([xc, xm, xd], axis=0)
    y0 = jnp.concatenate([yc, ym, yd], axis=0).reshape(N * 8, 16)

    agg0, deg = _sc_agg8(y0, idx8, dst2d)
    agg0 = agg0.reshape(NROWS, HIDDEN)[:N]
    deg = deg.reshape(NROWS, 32)[:N]

    x1, y1 = _layer_mid(x, agg0, deg, w0_top, b_mlp0, ln0_g, ln0_b,
                        w1_bot, 1000)

    agg1, = _sc_agg4(y1.reshape(N * 4, 16), idx4, dst2d)
    agg1 = agg1.reshape(NROWS, OUT)[:N]

    out = _layer_out(x1, agg1, deg, w1_top, b_mlp1, ln1_g, ln1_b, 1000)
    return out[:N_CARD], out[N_CARD:do], out[do:]


# speedup vs baseline: 2.7775x; 1.2465x over previous
import functools
import math

import jax
import jax.numpy as jnp
from jax import lax
from jax.experimental import pallas as pl
from jax.experimental.pallas import tpu as pltpu, tpu_sc as plsc

N_CARD = 50000
N_MERCH = 30000
N_DEV = 20000
N = N_CARD + N_MERCH + N_DEV
HIDDEN = 128
OUT = 64
LN_EPS = 1e-5

E = 2 * (150000 + 100000 + 50000)
EB = 128
NSC = 2
NTILE = 16
RPT = 320
NSEG = 5
RSEG = RPT // NSEG
NB = RPT * NTILE
EP = NB * EB
NDUMP = 128
NROWS = 100224
ZROWS = NROWS // NTILE
ZBUF = 232

_SQRT_HALF = 1.0 / math.sqrt(2.0)


def _enc_body(x_ref, w_ref, b_ref, wb_ref, x_out, y_out):
    xb = jnp.maximum(
        jnp.dot(x_ref[...], w_ref[...], preferred_element_type=jnp.float32)
        + b_ref[...], 0.0)
    x_out[...] = xb
    y_out[...] = jnp.dot(xb, wb_ref[...], preferred_element_type=jnp.float32)


def _encode(x, w, b, w_bot, rows, blk):
    grid = rows // blk
    fin = x.shape[1]
    fout = w_bot.shape[1]
    return pl.pallas_call(
        _enc_body,
        grid=(grid,),
        in_specs=[
            pl.BlockSpec((blk, fin), lambda i: (i, 0)),
            pl.BlockSpec((fin, HIDDEN), lambda i: (0, 0)),
            pl.BlockSpec((1, HIDDEN), lambda i: (0, 0)),
            pl.BlockSpec((HIDDEN, fout), lambda i: (0, 0)),
        ],
        out_specs=[
            pl.BlockSpec((blk, HIDDEN), lambda i: (i, 0)),
            pl.BlockSpec((blk, fout), lambda i: (i, 0)),
        ],
        out_shape=[
            jax.ShapeDtypeStruct((rows, HIDDEN), jnp.float32),
            jax.ShapeDtypeStruct((rows, fout), jnp.float32),
        ],
    )(x, w, b.reshape(1, HIDDEN), w_bot)


def _gelu_ln(h, g_ref, be_ref):
    h = 0.5 * h * (1.0 + lax.erf(h * _SQRT_HALF))
    mu = jnp.mean(h, axis=-1, keepdims=True)
    var = jnp.mean((h - mu) ** 2, axis=-1, keepdims=True)
    return (h - mu) * lax.rsqrt(var + LN_EPS) * g_ref[...] + be_ref[...]


def _layer_mid_body(x_ref, agg_ref, deg_ref, wt_ref, b_ref, g_ref, be_ref,
                    wb_ref, x1_out, y1_out):
    deg = deg_ref[...]
    degs = deg[:, 0:1] + deg[:, 16:17]
    inv = 1.0 / jnp.maximum(degs, 1.0)
    h = (jnp.dot(x_ref[...], wt_ref[...], preferred_element_type=jnp.float32)
         + agg_ref[...] * inv + b_ref[...])
    x1 = _gelu_ln(h, g_ref, be_ref)
    x1_out[...] = x1
    y1_out[...] = jnp.dot(x1, wb_ref[...], preferred_element_type=jnp.float32)


def _layer_mid(x, agg, deg, w_top, b, g, be, w_bot, blk):
    grid = N // blk
    return pl.pallas_call(
        _layer_mid_body,
        grid=(grid,),
        in_specs=[
            pl.BlockSpec((blk, HIDDEN), lambda i: (i, 0)),
            pl.BlockSpec((blk, HIDDEN), lambda i: (i, 0)),
            pl.BlockSpec((blk, 32), lambda i: (i, 0)),
            pl.BlockSpec((HIDDEN, HIDDEN), lambda i: (0, 0)),
            pl.BlockSpec((1, HIDDEN), lambda i: (0, 0)),
            pl.BlockSpec((1, HIDDEN), lambda i: (0, 0)),
            pl.BlockSpec((1, HIDDEN), lambda i: (0, 0)),
            pl.BlockSpec((HIDDEN, OUT), lambda i: (0, 0)),
        ],
        out_specs=[
            pl.BlockSpec((blk, HIDDEN), lambda i: (i, 0)),
            pl.BlockSpec((blk, OUT), lambda i: (i, 0)),
        ],
        out_shape=[
            jax.ShapeDtypeStruct((N, HIDDEN), jnp.float32),
            jax.ShapeDtypeStruct((N, OUT), jnp.float32),
        ],
    )(x, agg, deg, w_top, b.reshape(1, -1), g.reshape(1, -1),
      be.reshape(1, -1), w_bot)


def _layer_out_body(x_ref, agg_ref, deg_ref, wt_ref, b_ref, g_ref, be_ref,
                    out_ref):
    deg = deg_ref[...]
    degs = deg[:, 0:1] + deg[:, 16:17]
    inv = 1.0 / jnp.maximum(degs, 1.0)
    h = (jnp.dot(x_ref[...], wt_ref[...], preferred_element_type=jnp.float32)
         + agg_ref[...] * inv + b_ref[...])
    out_ref[...] = _gelu_ln(h, g_ref, be_ref)


def _layer_out(x, agg, deg, w_top, b, g, be, blk):
    grid = N // blk
    return pl.pallas_call(
        _layer_out_body,
        grid=(grid,),
        in_specs=[
            pl.BlockSpec((blk, HIDDEN), lambda i: (i, 0)),
            pl.BlockSpec((blk, OUT), lambda i: (i, 0)),
            pl.BlockSpec((blk, 32), lambda i: (i, 0)),
            pl.BlockSpec((HIDDEN, OUT), lambda i: (0, 0)),
            pl.BlockSpec((1, OUT), lambda i: (0, 0)),
            pl.BlockSpec((1, OUT), lambda i: (0, 0)),
            pl.BlockSpec((1, OUT), lambda i: (0, 0)),
        ],
        out_specs=pl.BlockSpec((blk, OUT), lambda i: (i, 0)),
        out_shape=jax.ShapeDtypeStruct((N, OUT), jnp.float32),
    )(x, agg, deg, w_top, b.reshape(1, -1), g.reshape(1, -1),
      be.reshape(1, -1))


def _zero_slice(acc, zbuf, sid):
    base = sid * ZROWS
    for t in range(ZROWS // ZBUF):
        pltpu.sync_copy(zbuf, acc.at[pl.ds(base + t * ZBUF, ZBUF)])
    rem = ZROWS % ZBUF
    if rem:
        pltpu.sync_copy(zbuf.at[pl.ds(0, rem)],
                        acc.at[pl.ds(base + (ZROWS // ZBUF) * ZBUF, rem)])


def _make_sc_agg(nchunk, with_deg):
    kpc = nchunk // NSC
    out_type = [jax.ShapeDtypeStruct((NROWS, nchunk, 16), jnp.float32)]
    if with_deg:
        out_type.append(jax.ShapeDtypeStruct((NROWS, 2, 16), jnp.float32))

    mesh = plsc.VectorSubcoreMesh(core_axis_name="c", subcore_axis_name="s")

    @functools.partial(
        pl.kernel,
        out_type=out_type,
        mesh=mesh,
        compiler_params=pltpu.CompilerParams(use_tc_tiling_on_sc=False),
        scratch_types=[
            pltpu.VMEM_SHARED((NROWS, 16), jnp.float32),
            pltpu.VMEM((RSEG, EB), jnp.int32),
            pltpu.VMEM((RSEG, EB), jnp.int32),
            pltpu.VMEM((EB, 16), jnp.float32),
            pltpu.VMEM((EB, 16), jnp.float32),
            pltpu.VMEM((ZBUF, 16), jnp.float32),
            pltpu.SemaphoreType.DMA,
            pltpu.SemaphoreType.DMA,
        ],
    )
    def sc_agg(table, idx, dst, *rest):
        if with_deg:
            agg_out, deg_out, acc, idxv, dstv, rows, rows1, zbuf, \
                sem0, sem1 = rest
        else:
            agg_out, acc, idxv, dstv, rows, rows1, zbuf, sem0, sem1 = rest
        cid = lax.axis_index("c")
        sid = lax.axis_index("s")

        def _zb(i, _):
            zbuf[i, :] = jnp.zeros((16,), jnp.float32)
            return 0
        lax.fori_loop(0, ZBUF, _zb, 0)

        def _chunk(k):
            _zero_slice(acc, zbuf, sid)
            plsc.subcore_barrier()
            for s in range(NSEG):
                base = s * (NB // NSEG) + sid * RSEG
                pltpu.sync_copy(idx.at[pl.ds(k * NB + base, RSEG)], idxv)
                pltpu.sync_copy(dst.at[pl.ds(base, RSEG)], dstv)

                pltpu.async_copy(table.at[idxv.at[0]], rows, sem0)

                def _edge(j, _):
                    even = (j % 2) == 0
                    last = j == (RSEG - 1)

                    @pl.when(jnp.logical_and(even, jnp.logical_not(last)))
                    def _():
                        pltpu.async_copy(table.at[idxv.at[j + 1]], rows1,
                                         sem1)

                    @pl.when(jnp.logical_and(jnp.logical_not(even),
                                             jnp.logical_not(last)))
                    def _():
                        pltpu.async_copy(table.at[idxv.at[j + 1]], rows,
                                         sem0)

                    @pl.when(even)
                    def _():
                        pltpu.make_async_copy(table.at[idxv.at[j]], rows,
                                              sem0).wait()
                        pltpu.sync_copy(rows, acc.at[dstv.at[j]], add=True)

                    @pl.when(jnp.logical_not(even))
                    def _():
                        pltpu.make_async_copy(table.at[idxv.at[j]], rows1,
                                              sem1).wait()
                        pltpu.sync_copy(rows1, acc.at[dstv.at[j]], add=True)
                    return 0
                lax.fori_loop(0, RSEG, _edge, 0)
            plsc.subcore_barrier()
            pltpu.sync_copy(acc.at[pl.ds(sid * ZROWS, ZROWS)],
                            agg_out.at[pl.ds(sid * ZROWS, ZROWS), k])
            plsc.subcore_barrier()

        for cc in range(NSC):
            @pl.when(cid == cc)
            def _core_chunks(cc=cc):
                for kk in range(kpc):
                    _chunk(cc * kpc + kk)

        if with_deg:
            def _ob(i, _):
                rows[i, :] = jnp.ones((16,), jnp.float32)
                return 0
            lax.fori_loop(0, EB, _ob, 0)
            _zero_slice(acc, zbuf, sid)
            plsc.subcore_barrier()
            half = RSEG // NSC

            def _deg_segs(cc):
                for s in range(NSEG):
                    base = s * (NB // NSEG) + sid * RSEG
                    pltpu.sync_copy(dst.at[pl.ds(base, RSEG)], dstv)

                    def _dedge(j, _):
                        pltpu.sync_copy(rows, acc.at[dstv.at[j]], add=True)
                        return 0
                    lax.fori_loop(cc * half, (cc + 1) * half, _dedge, 0)
                plsc.subcore_barrier()
                pltpu.sync_copy(acc.at[pl.ds(sid * ZROWS, ZROWS)],
                                deg_out.at[pl.ds(sid * ZROWS, ZROWS), cc])

            for cc in range(NSC):
                @pl.when(cid == cc)
                def _core_deg(cc=cc):
                    _deg_segs(cc)

    return sc_agg


_make_sc_agg = functools.lru_cache(None)(_make_sc_agg)


def _sc_agg8(*args):
    return _make_sc_agg(8, True)(*args)


def _sc_agg4(*args):
    return _make_sc_agg(4, False)(*args)


def kernel(x_card, x_merchant, x_device, edge_pays, edge_uses, edge_seen_at,
           W_card, b_card, W_merchant, b_merchant, W_device, b_device,
           W_mlp0, b_mlp0, ln0_g, ln0_b, W_mlp1, b_mlp1, ln1_g, ln1_b):
    mo = N_CARD
    do = N_CARD + N_MERCH
    src = jnp.concatenate([
        edge_pays[0], edge_pays[1] + mo, edge_uses[0], edge_uses[1] + do,
        edge_seen_at[0] + do, edge_seen_at[1] + mo])
    dst = jnp.concatenate([
        edge_pays[1] + mo, edge_pays[0], edge_uses[1] + do, edge_uses[0],
        edge_seen_at[1] + mo, edge_seen_at[0] + do])
    npad = EP - E
    pad_lanes = jnp.arange(npad, dtype=src.dtype)
    src_p = jnp.concatenate([src, pad_lanes % NDUMP])
    dst_p = jnp.concatenate([dst, N + (pad_lanes % NDUMP)])
    dst2d = dst_p.reshape(NB, EB)
    ks8 = jnp.arange(8, dtype=src.dtype)[:, None]
    idx8 = (src_p[None, :] * 8 + ks8).reshape(8 * NB, EB)
    ks4 = jnp.arange(4, dtype=src.dtype)[:, None]
    idx4 = (src_p[None, :] * 4 + ks4).reshape(4 * NB, EB)

    w0_top, w0_bot = W_mlp0[:HIDDEN], W_mlp0[HIDDEN:]
    w1_top, w1_bot = W_mlp1[:HIDDEN], W_mlp1[HIDDEN:]

    xc, yc = _encode(x_card, W_card, b_card, w0_bot, N_CARD, 1000)
    xm, ym = _encode(x_merchant, W_merchant, b_merchant, w0_bot, N_MERCH, 1000)
    xd, yd = _encode(x_device, W_device, b_device, w0_bot, N_DEV, 1000)
    x = jnp.concatenate([xc, xm, xd], axis=0)
    y0 = jnp.concatenate([yc, ym, yd], axis=0).reshape(N * 8, 16)

    agg0, deg = _sc_agg8(y0, idx8, dst2d)
    agg0 = agg0.reshape(NROWS, HIDDEN)[:N]
    deg = deg.reshape(NROWS, 32)[:N]

    x1, y1 = _layer_mid(x, agg0, deg, w0_top, b_mlp0, ln0_g, ln0_b,
                        w1_bot, 1000)

    agg1, = _sc_agg4(y1.reshape(N * 4, 16), idx4, dst2d)
    agg1 = agg1.reshape(NROWS, OUT)[:N]

    out = _layer_out(x1, agg1, deg, w1_top, b_mlp1, ln1_g, ln1_b, 1000)
    return out[:N_CARD], out[N_CARD:do], out[do:]


# speedup vs baseline: 4.4669x; 1.6082x over previous
import functools
import math

import jax
import jax.numpy as jnp
from jax import lax
from jax.experimental import pallas as pl
from jax.experimental.pallas import tpu as pltpu, tpu_sc as plsc

N_CARD = 50000
N_MERCH = 30000
N_DEV = 20000
N = N_CARD + N_MERCH + N_DEV
HIDDEN = 128
OUT = 64
LN_EPS = 1e-5

E = 2 * (150000 + 100000 + 50000)
EB = 128
NSC = 2
NTILE = 16
RPT = 320
NSEG = 8
RSEG = RPT // NSEG
NB = RPT * NTILE
EP = NB * EB
NDUMP = 128
NROWS = 100352
ZROWS = NROWS // NTILE
ZBUF = 448

_SQRT_HALF = 1.0 / math.sqrt(2.0)


def _enc_body(x_ref, w_ref, b_ref, wb_ref, x_out, y_out):
    xb = jnp.maximum(
        jnp.dot(x_ref[...], w_ref[...], preferred_element_type=jnp.float32)
        + b_ref[...], 0.0)
    x_out[...] = xb
    y_out[...] = jnp.dot(xb, wb_ref[...], preferred_element_type=jnp.float32)


def _encode(x, w, b, w_bot, rows, blk):
    grid = rows // blk
    fin = x.shape[1]
    fout = w_bot.shape[1]
    return pl.pallas_call(
        _enc_body,
        grid=(grid,),
        in_specs=[
            pl.BlockSpec((blk, fin), lambda i: (i, 0)),
            pl.BlockSpec((fin, HIDDEN), lambda i: (0, 0)),
            pl.BlockSpec((1, HIDDEN), lambda i: (0, 0)),
            pl.BlockSpec((HIDDEN, fout), lambda i: (0, 0)),
        ],
        out_specs=[
            pl.BlockSpec((blk, HIDDEN), lambda i: (i, 0)),
            pl.BlockSpec((blk, fout), lambda i: (i, 0)),
        ],
        out_shape=[
            jax.ShapeDtypeStruct((rows, HIDDEN), jnp.float32),
            jax.ShapeDtypeStruct((rows, fout), jnp.float32),
        ],
    )(x, w, b.reshape(1, HIDDEN), w_bot)


def _gelu_ln(h, g_ref, be_ref):
    h = 0.5 * h * (1.0 + lax.erf(h * _SQRT_HALF))
    mu = jnp.mean(h, axis=-1, keepdims=True)
    var = jnp.mean((h - mu) ** 2, axis=-1, keepdims=True)
    return (h - mu) * lax.rsqrt(var + LN_EPS) * g_ref[...] + be_ref[...]


def _layer_mid_body(x_ref, agg_ref, deg_ref, wt_ref, b_ref, g_ref, be_ref,
                    wb_ref, x1_out, y1_out):
    deg = deg_ref[...]
    degs = deg[:, 0:1] + deg[:, 16:17]
    inv = 1.0 / jnp.maximum(degs, 1.0)
    h = (jnp.dot(x_ref[...], wt_ref[...], preferred_element_type=jnp.float32)
         + agg_ref[...] * inv + b_ref[...])
    x1 = _gelu_ln(h, g_ref, be_ref)
    x1_out[...] = x1
    y1_out[...] = jnp.dot(x1, wb_ref[...], preferred_element_type=jnp.float32)


def _layer_mid(x, agg, deg, w_top, b, g, be, w_bot, blk):
    grid = N // blk
    return pl.pallas_call(
        _layer_mid_body,
        grid=(grid,),
        in_specs=[
            pl.BlockSpec((blk, HIDDEN), lambda i: (i, 0)),
            pl.BlockSpec((blk, HIDDEN), lambda i: (i, 0)),
            pl.BlockSpec((blk, 32), lambda i: (i, 0)),
            pl.BlockSpec((HIDDEN, HIDDEN), lambda i: (0, 0)),
            pl.BlockSpec((1, HIDDEN), lambda i: (0, 0)),
            pl.BlockSpec((1, HIDDEN), lambda i: (0, 0)),
            pl.BlockSpec((1, HIDDEN), lambda i: (0, 0)),
            pl.BlockSpec((HIDDEN, OUT), lambda i: (0, 0)),
        ],
        out_specs=[
            pl.BlockSpec((blk, HIDDEN), lambda i: (i, 0)),
            pl.BlockSpec((blk, OUT), lambda i: (i, 0)),
        ],
        out_shape=[
            jax.ShapeDtypeStruct((N, HIDDEN), jnp.float32),
            jax.ShapeDtypeStruct((N, OUT), jnp.float32),
        ],
    )(x, agg, deg, w_top, b.reshape(1, -1), g.reshape(1, -1),
      be.reshape(1, -1), w_bot)


def _layer_out_body(x_ref, agg_ref, deg_ref, wt_ref, b_ref, g_ref, be_ref,
                    out_ref):
    deg = deg_ref[...]
    degs = deg[:, 0:1] + deg[:, 16:17]
    inv = 1.0 / jnp.maximum(degs, 1.0)
    h = (jnp.dot(x_ref[...], wt_ref[...], preferred_element_type=jnp.float32)
         + agg_ref[...] * inv + b_ref[...])
    out_ref[...] = _gelu_ln(h, g_ref, be_ref)


def _layer_out(x, agg, deg, w_top, b, g, be, blk):
    grid = N // blk
    return pl.pallas_call(
        _layer_out_body,
        grid=(grid,),
        in_specs=[
            pl.BlockSpec((blk, HIDDEN), lambda i: (i, 0)),
            pl.BlockSpec((blk, OUT), lambda i: (i, 0)),
            pl.BlockSpec((blk, 32), lambda i: (i, 0)),
            pl.BlockSpec((HIDDEN, OUT), lambda i: (0, 0)),
            pl.BlockSpec((1, OUT), lambda i: (0, 0)),
            pl.BlockSpec((1, OUT), lambda i: (0, 0)),
            pl.BlockSpec((1, OUT), lambda i: (0, 0)),
        ],
        out_specs=pl.BlockSpec((blk, OUT), lambda i: (i, 0)),
        out_shape=jax.ShapeDtypeStruct((N, OUT), jnp.float32),
    )(x, agg, deg, w_top, b.reshape(1, -1), g.reshape(1, -1),
      be.reshape(1, -1))


def _zero_slice(acc, zbuf, sid):
    base = sid * ZROWS
    for t in range(ZROWS // ZBUF):
        pltpu.sync_copy(zbuf, acc.at[pl.ds(base + t * ZBUF, ZBUF)])
    rem = ZROWS % ZBUF
    if rem:
        pltpu.sync_copy(zbuf.at[pl.ds(0, rem)],
                        acc.at[pl.ds(base + (ZROWS // ZBUF) * ZBUF, rem)])


def _make_sc_agg(nchunk, with_deg):
    kpc = nchunk // NSC
    shift = nchunk.bit_length() - 1
    out_type = [jax.ShapeDtypeStruct((NROWS, nchunk * 16), jnp.float32)]
    if with_deg:
        out_type.append(jax.ShapeDtypeStruct((NROWS, 32), jnp.float32))

    mesh = plsc.VectorSubcoreMesh(core_axis_name="c", subcore_axis_name="s")

    @functools.partial(
        pl.kernel,
        out_type=out_type,
        mesh=mesh,
        compiler_params=pltpu.CompilerParams(use_tc_tiling_on_sc=False),
        scratch_types=[
            pltpu.VMEM_SHARED((NROWS, 16), jnp.float32),
            pltpu.VMEM((RSEG, EB), jnp.int32),
            pltpu.VMEM((RSEG, EB), jnp.int32),
            pltpu.VMEM((RSEG, EB), jnp.int32),
            pltpu.VMEM((EB, 16), jnp.float32),
            pltpu.VMEM((EB, 16), jnp.float32),
            pltpu.VMEM((ZBUF, 16), jnp.float32),
            pltpu.SemaphoreType.DMA,
            pltpu.SemaphoreType.DMA,
        ],
    )
    def sc_agg(table, src, dst, *rest):
        if with_deg:
            agg_out, deg_out, acc, idxv, srcv, dstv, rows, rows1, zbuf, \
                sem0, sem1 = rest
        else:
            agg_out, acc, idxv, srcv, dstv, rows, rows1, zbuf, \
                sem0, sem1 = rest
        cid = lax.axis_index("c")
        sid = lax.axis_index("s")

        def _zb(i, _):
            zbuf[i, :] = jnp.zeros((16,), jnp.float32)
            return 0
        lax.fori_loop(0, ZBUF, _zb, 0)

        def _chunk(k):
            _zero_slice(acc, zbuf, sid)
            plsc.subcore_barrier()
            for s in range(NSEG):
                base = s * (NB // NSEG) + sid * RSEG
                pltpu.sync_copy(src.at[pl.ds(base, RSEG)], srcv)
                pltpu.sync_copy(dst.at[pl.ds(base, RSEG)], dstv)

                def _mkidx(r, _):
                    for c in range(EB // 16):
                        sl = (r, pl.ds(c * 16, 16))
                        idxv[sl] = (srcv[sl] << shift) + k
                    return 0
                lax.fori_loop(0, RSEG, _mkidx, 0)

                pltpu.async_copy(table.at[idxv.at[0]], rows, sem0)

                def _edge(j, _):
                    even = (j % 2) == 0
                    last = j == (RSEG - 1)

                    @pl.when(jnp.logical_and(even, jnp.logical_not(last)))
                    def _():
                        pltpu.async_copy(table.at[idxv.at[j + 1]], rows1,
                                         sem1)

                    @pl.when(jnp.logical_and(jnp.logical_not(even),
                                             jnp.logical_not(last)))
                    def _():
                        pltpu.async_copy(table.at[idxv.at[j + 1]], rows,
                                         sem0)

                    @pl.when(even)
                    def _():
                        pltpu.make_async_copy(table.at[idxv.at[j]], rows,
                                              sem0).wait()
                        pltpu.sync_copy(rows, acc.at[dstv.at[j]], add=True)

                    @pl.when(jnp.logical_not(even))
                    def _():
                        pltpu.make_async_copy(table.at[idxv.at[j]], rows1,
                                              sem1).wait()
                        pltpu.sync_copy(rows1, acc.at[dstv.at[j]], add=True)
                    return 0
                lax.fori_loop(0, RSEG, _edge, 0)
            plsc.subcore_barrier()
            pltpu.sync_copy(acc.at[pl.ds(sid * ZROWS, ZROWS)],
                            agg_out.at[pl.ds(sid * ZROWS, ZROWS),
                                       pl.ds(16 * k, 16)])
            plsc.subcore_barrier()

        for cc in range(NSC):
            @pl.when(cid == cc)
            def _core_chunks(cc=cc):
                for kk in range(kpc):
                    _chunk(cc * kpc + kk)

        if with_deg:
            def _ob(i, _):
                rows[i, :] = jnp.ones((16,), jnp.float32)
                return 0
            lax.fori_loop(0, EB, _ob, 0)
            _zero_slice(acc, zbuf, sid)
            plsc.subcore_barrier()
            half = RSEG // NSC

            def _deg_segs(cc):
                for s in range(NSEG):
                    base = s * (NB // NSEG) + sid * RSEG
                    pltpu.sync_copy(dst.at[pl.ds(base, RSEG)], dstv)

                    def _dedge(j, _):
                        pltpu.sync_copy(rows, acc.at[dstv.at[j]], add=True)
                        return 0
                    lax.fori_loop(cc * half, (cc + 1) * half, _dedge, 0)
                plsc.subcore_barrier()
                pltpu.sync_copy(acc.at[pl.ds(sid * ZROWS, ZROWS)],
                                deg_out.at[pl.ds(sid * ZROWS, ZROWS),
                                           pl.ds(16 * cc, 16)])

            for cc in range(NSC):
                @pl.when(cid == cc)
                def _core_deg(cc=cc):
                    _deg_segs(cc)

    return sc_agg


_make_sc_agg = functools.lru_cache(None)(_make_sc_agg)


def _sc_agg8(*args):
    return _make_sc_agg(8, True)(*args)


def _sc_agg4(*args):
    return _make_sc_agg(4, False)(*args)


def kernel(x_card, x_merchant, x_device, edge_pays, edge_uses, edge_seen_at,
           W_card, b_card, W_merchant, b_merchant, W_device, b_device,
           W_mlp0, b_mlp0, ln0_g, ln0_b, W_mlp1, b_mlp1, ln1_g, ln1_b):
    mo = N_CARD
    do = N_CARD + N_MERCH
    src = jnp.concatenate([
        edge_pays[0], edge_pays[1] + mo, edge_uses[0], edge_uses[1] + do,
        edge_seen_at[0] + do, edge_seen_at[1] + mo])
    dst = jnp.concatenate([
        edge_pays[1] + mo, edge_pays[0], edge_uses[1] + do, edge_uses[0],
        edge_seen_at[1] + mo, edge_seen_at[0] + do])
    npad = EP - E
    pad_lanes = jnp.arange(npad, dtype=src.dtype)
    src_p = jnp.concatenate([src, pad_lanes % NDUMP])
    dst_p = jnp.concatenate([dst, N + (pad_lanes % NDUMP)])
    src2d = src_p.reshape(NB, EB)
    dst2d = dst_p.reshape(NB, EB)

    w0_top, w0_bot = W_mlp0[:HIDDEN], W_mlp0[HIDDEN:]
    w1_top, w1_bot = W_mlp1[:HIDDEN], W_mlp1[HIDDEN:]

    xc, yc = _encode(x_card, W_card, b_card, w0_bot, N_CARD, 1000)
    xm, ym = _encode(x_merchant, W_merchant, b_merchant, w0_bot, N_MERCH, 1000)
    xd, yd = _encode(x_device, W_device, b_device, w0_bot, N_DEV, 1000)
    x = jnp.concatenate([xc, xm, xd], axis=0)
    y0 = jnp.concatenate([yc, ym, yd], axis=0).reshape(N * 8, 16)

    agg0, deg = _sc_agg8(y0, src2d, dst2d)

    x1, y1 = _layer_mid(x, agg0, deg, w0_top, b_mlp0, ln0_g, ln0_b,
                        w1_bot, 1000)

    agg1, = _sc_agg4(y1.reshape(N * 4, 16), src2d, dst2d)

    out = _layer_out(x1, agg1, deg, w1_top, b_mlp1, ln1_g, ln1_b, 1000)
    return out[:N_CARD], out[N_CARD:do], out[do:]


# speedup vs baseline: 5.2782x; 1.1816x over previous
import functools
import math

import jax
import jax.numpy as jnp
from jax import lax
from jax.experimental import pallas as pl
from jax.experimental.pallas import tpu as pltpu, tpu_sc as plsc

N_CARD = 50000
N_MERCH = 30000
N_DEV = 20000
N = N_CARD + N_MERCH + N_DEV
HIDDEN = 128
OUT = 64
LN_EPS = 1e-5

E = 2 * (150000 + 100000 + 50000)
EB = 128
NSC = 2
NTILE = 16
RPT = 320
NSEG = 8
RSEG = RPT // NSEG
G = 320
NGRP = RSEG * EB // G
NPAIR = NGRP // 2
NB = RPT * NTILE
EP = NB * EB
NDUMP = 128
NROWS = 100352
ZROWS = NROWS // NTILE
ZBUF = 224

_SQRT_HALF = 1.0 / math.sqrt(2.0)


def _enc_body(x_ref, w_ref, b_ref, wb_ref, x_out, y_out):
    xb = jnp.maximum(
        jnp.dot(x_ref[...], w_ref[...], preferred_element_type=jnp.float32)
        + b_ref[...], 0.0)
    x_out[...] = xb
    y_out[...] = jnp.dot(xb, wb_ref[...], preferred_element_type=jnp.float32)


def _encode(x, w, b, w_bot, rows, blk):
    grid = rows // blk
    fin = x.shape[1]
    fout = w_bot.shape[1]
    return pl.pallas_call(
        _enc_body,
        grid=(grid,),
        in_specs=[
            pl.BlockSpec((blk, fin), lambda i: (i, 0)),
            pl.BlockSpec((fin, HIDDEN), lambda i: (0, 0)),
            pl.BlockSpec((1, HIDDEN), lambda i: (0, 0)),
            pl.BlockSpec((HIDDEN, fout), lambda i: (0, 0)),
        ],
        out_specs=[
            pl.BlockSpec((blk, HIDDEN), lambda i: (i, 0)),
            pl.BlockSpec((blk, fout), lambda i: (i, 0)),
        ],
        out_shape=[
            jax.ShapeDtypeStruct((rows, HIDDEN), jnp.float32),
            jax.ShapeDtypeStruct((rows, fout), jnp.float32),
        ],
    )(x, w, b.reshape(1, HIDDEN), w_bot)


def _gelu_ln(h, g_ref, be_ref):
    h = 0.5 * h * (1.0 + lax.erf(h * _SQRT_HALF))
    mu = jnp.mean(h, axis=-1, keepdims=True)
    var = jnp.mean((h - mu) ** 2, axis=-1, keepdims=True)
    return (h - mu) * lax.rsqrt(var + LN_EPS) * g_ref[...] + be_ref[...]


def _layer_mid_body(x_ref, agg_ref, deg_ref, wt_ref, b_ref, g_ref, be_ref,
                    wb_ref, x1_out, y1_out):
    deg = deg_ref[...]
    degs = deg[:, 0:1] + deg[:, 16:17]
    inv = 1.0 / jnp.maximum(degs, 1.0)
    h = (jnp.dot(x_ref[...], wt_ref[...], preferred_element_type=jnp.float32)
         + agg_ref[...] * inv + b_ref[...])
    x1 = _gelu_ln(h, g_ref, be_ref)
    x1_out[...] = x1
    y1_out[...] = jnp.dot(x1, wb_ref[...], preferred_element_type=jnp.float32)


def _layer_mid(x, agg, deg, w_top, b, g, be, w_bot, blk):
    grid = N // blk
    return pl.pallas_call(
        _layer_mid_body,
        grid=(grid,),
        in_specs=[
            pl.BlockSpec((blk, HIDDEN), lambda i: (i, 0)),
            pl.BlockSpec((blk, HIDDEN), lambda i: (i, 0)),
            pl.BlockSpec((blk, 32), lambda i: (i, 0)),
            pl.BlockSpec((HIDDEN, HIDDEN), lambda i: (0, 0)),
            pl.BlockSpec((1, HIDDEN), lambda i: (0, 0)),
            pl.BlockSpec((1, HIDDEN), lambda i: (0, 0)),
            pl.BlockSpec((1, HIDDEN), lambda i: (0, 0)),
            pl.BlockSpec((HIDDEN, OUT), lambda i: (0, 0)),
        ],
        out_specs=[
            pl.BlockSpec((blk, HIDDEN), lambda i: (i, 0)),
            pl.BlockSpec((blk, OUT), lambda i: (i, 0)),
        ],
        out_shape=[
            jax.ShapeDtypeStruct((N, HIDDEN), jnp.float32),
            jax.ShapeDtypeStruct((N, OUT), jnp.float32),
        ],
    )(x, agg, deg, w_top, b.reshape(1, -1), g.reshape(1, -1),
      be.reshape(1, -1), w_bot)


def _layer_out_body(x_ref, agg_ref, deg_ref, wt_ref, b_ref, g_ref, be_ref,
                    out_ref):
    deg = deg_ref[...]
    degs = deg[:, 0:1] + deg[:, 16:17]
    inv = 1.0 / jnp.maximum(degs, 1.0)
    h = (jnp.dot(x_ref[...], wt_ref[...], preferred_element_type=jnp.float32)
         + agg_ref[...] * inv + b_ref[...])
    out_ref[...] = _gelu_ln(h, g_ref, be_ref)


def _layer_out(x, agg, deg, w_top, b, g, be, blk):
    grid = N // blk
    return pl.pallas_call(
        _layer_out_body,
        grid=(grid,),
        in_specs=[
            pl.BlockSpec((blk, HIDDEN), lambda i: (i, 0)),
            pl.BlockSpec((blk, OUT), lambda i: (i, 0)),
            pl.BlockSpec((blk, 32), lambda i: (i, 0)),
            pl.BlockSpec((HIDDEN, OUT), lambda i: (0, 0)),
            pl.BlockSpec((1, OUT), lambda i: (0, 0)),
            pl.BlockSpec((1, OUT), lambda i: (0, 0)),
            pl.BlockSpec((1, OUT), lambda i: (0, 0)),
        ],
        out_specs=pl.BlockSpec((blk, OUT), lambda i: (i, 0)),
        out_shape=jax.ShapeDtypeStruct((N, OUT), jnp.float32),
    )(x, agg, deg, w_top, b.reshape(1, -1), g.reshape(1, -1),
      be.reshape(1, -1))


def _zero_slice(acc, zbuf, sid):
    base = sid * ZROWS

    def _z(t, _):
        pltpu.sync_copy(zbuf, acc.at[pl.ds(base + t * ZBUF, ZBUF)])
        return 0
    lax.fori_loop(0, ZROWS // ZBUF, _z, 0)


def _make_sc_agg(nchunk, with_deg):
    kpc = nchunk // NSC
    shift = nchunk.bit_length() - 1
    out_type = [jax.ShapeDtypeStruct((NROWS, nchunk * 16), jnp.float32)]
    if with_deg:
        out_type.append(jax.ShapeDtypeStruct((NROWS, 32), jnp.float32))

    mesh = plsc.VectorSubcoreMesh(core_axis_name="c", subcore_axis_name="s")

    @functools.partial(
        pl.kernel,
        out_type=out_type,
        mesh=mesh,
        compiler_params=pltpu.CompilerParams(use_tc_tiling_on_sc=False),
        scratch_types=[
            pltpu.VMEM_SHARED((NROWS, 16), jnp.float32),
            pltpu.VMEM((RSEG * EB,), jnp.int32),
            pltpu.VMEM((RSEG * EB,), jnp.int32),
            pltpu.VMEM((RSEG * EB,), jnp.int32),
            pltpu.VMEM((G, 16), jnp.float32),
            pltpu.VMEM((G, 16), jnp.float32),
            pltpu.VMEM((ZBUF, 16), jnp.float32),
            pltpu.SemaphoreType.DMA,
            pltpu.SemaphoreType.DMA,
        ],
    )
    def sc_agg(table, src, dst, *rest):
        if with_deg:
            agg_out, deg_out, acc, idxv, srcv, dstv, rows, rows1, zbuf, \
                sem0, sem1 = rest
        else:
            agg_out, acc, idxv, srcv, dstv, rows, rows1, zbuf, \
                sem0, sem1 = rest
        cid = lax.axis_index("c")
        sid = lax.axis_index("s")

        def _zb(i, _):
            zbuf[i, :] = jnp.zeros((16,), jnp.float32)
            return 0
        lax.fori_loop(0, ZBUF, _zb, 0)

        def _chunk(k):
            _zero_slice(acc, zbuf, sid)
            plsc.subcore_barrier()
            for s in range(NSEG):
                base = s * (NB // NSEG) + sid * RSEG
                pltpu.sync_copy(src.at[pl.ds(base * EB, RSEG * EB)], srcv)
                pltpu.sync_copy(dst.at[pl.ds(base * EB, RSEG * EB)], dstv)

                def _mkidx(v, _):
                    sl = pl.ds(v * 16, 16)
                    idxv[sl] = (srcv[sl] << shift) + k
                    return 0
                lax.fori_loop(0, RSEG * EB // 16, _mkidx, 0)

                pltpu.async_copy(table.at[idxv.at[pl.ds(0, G)]], rows, sem0)

                def _scat(j, buf):
                    pltpu.sync_copy(buf, acc.at[dstv.at[pl.ds(j * G, G)]],
                                    add=True)

                def _pair(p, _):
                    j0 = 2 * p
                    pltpu.async_copy(
                        table.at[idxv.at[pl.ds((j0 + 1) * G, G)]],
                        rows1, sem1)
                    pltpu.make_async_copy(table.at[idxv.at[pl.ds(0, G)]],
                                          rows, sem0).wait()
                    _scat(j0, rows)

                    @pl.when(p < NPAIR - 1)
                    def _():
                        pltpu.async_copy(
                            table.at[idxv.at[pl.ds((j0 + 2) * G, G)]],
                            rows, sem0)
                    pltpu.make_async_copy(table.at[idxv.at[pl.ds(0, G)]],
                                          rows1, sem1).wait()
                    _scat(j0 + 1, rows1)
                    return 0
                lax.fori_loop(0, NPAIR, _pair, 0)
            plsc.subcore_barrier()
            pltpu.sync_copy(acc.at[pl.ds(sid * ZROWS, ZROWS)],
                            agg_out.at[pl.ds(sid * ZROWS, ZROWS),
                                       pl.ds(16 * k, 16)])
            plsc.subcore_barrier()

        for cc in range(NSC):
            @pl.when(cid == cc)
            def _core_chunks(cc=cc):
                def _ck(kk, _):
                    _chunk(cc * kpc + kk)
                    return 0
                lax.fori_loop(0, kpc, _ck, 0)

        if with_deg:
            def _ob(i, _):
                rows[i, :] = jnp.ones((16,), jnp.float32)
                return 0
            lax.fori_loop(0, G, _ob, 0)
            _zero_slice(acc, zbuf, sid)
            plsc.subcore_barrier()
            half_edges = RSEG * EB // NSC

            def _deg_segs(cc):
                for s in range(NSEG):
                    base = s * (NB // NSEG) + sid * RSEG
                    pltpu.sync_copy(dst.at[pl.ds(base * EB, RSEG * EB)],
                                    dstv)

                    def _dedge(q, _):
                        pltpu.sync_copy(
                            rows,
                            acc.at[dstv.at[pl.ds(cc * half_edges + q * G,
                                                 G)]],
                            add=True)
                        return 0
                    lax.fori_loop(0, half_edges // G, _dedge, 0)
                plsc.subcore_barrier()
                pltpu.sync_copy(acc.at[pl.ds(sid * ZROWS, ZROWS)],
                                deg_out.at[pl.ds(sid * ZROWS, ZROWS),
                                           pl.ds(16 * cc, 16)])

            for cc in range(NSC):
                @pl.when(cid == cc)
                def _core_deg(cc=cc):
                    _deg_segs(cc)

    return sc_agg


_make_sc_agg = functools.lru_cache(None)(_make_sc_agg)


def _sc_agg8(*args):
    return _make_sc_agg(8, True)(*args)


def _sc_agg4(*args):
    return _make_sc_agg(4, False)(*args)


def kernel(x_card, x_merchant, x_device, edge_pays, edge_uses, edge_seen_at,
           W_card, b_card, W_merchant, b_merchant, W_device, b_device,
           W_mlp0, b_mlp0, ln0_g, ln0_b, W_mlp1, b_mlp1, ln1_g, ln1_b):
    mo = N_CARD
    do = N_CARD + N_MERCH
    src = jnp.concatenate([
        edge_pays[0], edge_pays[1] + mo, edge_uses[0], edge_uses[1] + do,
        edge_seen_at[0] + do, edge_seen_at[1] + mo])
    dst = jnp.concatenate([
        edge_pays[1] + mo, edge_pays[0], edge_uses[1] + do, edge_uses[0],
        edge_seen_at[1] + mo, edge_seen_at[0] + do])
    npad = EP - E
    pad_lanes = jnp.arange(npad, dtype=src.dtype)
    src_p = jnp.concatenate([src, pad_lanes % NDUMP])
    dst_p = jnp.concatenate([dst, N + (pad_lanes % NDUMP)])

    w0_top, w0_bot = W_mlp0[:HIDDEN], W_mlp0[HIDDEN:]
    w1_top, w1_bot = W_mlp1[:HIDDEN], W_mlp1[HIDDEN:]

    xc, yc = _encode(x_card, W_card, b_card, w0_bot, N_CARD, 1000)
    xm, ym = _encode(x_merchant, W_merchant, b_merchant, w0_bot, N_MERCH, 1000)
    xd, yd = _encode(x_device, W_device, b_device, w0_bot, N_DEV, 1000)
    x = jnp.concatenate([xc, xm, xd], axis=0)
    y0 = jnp.concatenate([yc, ym, yd], axis=0).reshape(N * 8, 16)

    agg0, deg = _sc_agg8(y0, src_p, dst_p)

    x1, y1 = _layer_mid(x, agg0, deg, w0_top, b_mlp0, ln0_g, ln0_b,
                        w1_bot, 1000)

    agg1, = _sc_agg4(y1.reshape(N * 4, 16), src_p, dst_p)

    out = _layer_out(x1, agg1, deg, w1_top, b_mlp1, ln1_g, ln1_b, 1000)
    return out[:N_CARD], out[N_CARD:do], out[do:]
